# baseline (speedup 1.0000x reference)
import math

import jax
import jax.numpy as jnp
from jax import lax
from jax.experimental import pallas as pl
from jax.experimental.pallas import tpu as pltpu
from jax.experimental.pallas import tpu_sc as plsc

EMBED_DIM = 64
SEQ_LEN = 200
BATCH = 1024
ROWS = BATCH * SEQ_LEN
NC, NS, LANES = 2, 16, 16
NW = NC * NS
RPW = ROWS // NW
CHUNK = 64
NCHUNK = RPW // CHUNK
NBUF = 4
PE_EXT = 256
SCALE = math.sqrt(EMBED_DIM)


def _positional_table():
    pe_len = SEQ_LEN * 2
    pos = jnp.arange(pe_len, dtype=jnp.float32)[:, None]
    i = jnp.arange(0, EMBED_DIM, 2, dtype=jnp.float32)[None, :]
    sin_part = jnp.sin(pos / jnp.power(10000.0, 2.0 * i / EMBED_DIM))
    cos_part = jnp.cos(pos / jnp.power(10000.0, 2.0 * (i + 1.0) / EMBED_DIM))
    pe = jnp.zeros((pe_len, EMBED_DIM), dtype=jnp.float32)
    pe = pe.at[:, 0::2].set(sin_part)
    pe = pe.at[:, 1::2].set(cos_part)
    pe = pe[:SEQ_LEN]
    return jnp.concatenate([pe, pe[: PE_EXT - SEQ_LEN]], axis=0)


def _sc_body(idx_hbm, pe_hbm, table_hbm, out_hbm,
             idx_v, pair_v, par_v, pe_v, *rest):
    bufs = rest[:NBUF]
    obufs = rest[NBUF:2 * NBUF]
    sg = rest[2 * NBUF:3 * NBUF]
    so = rest[3 * NBUF:]
    cid = lax.axis_index("c")
    sid = lax.axis_index("s")
    wid = sid * NC + cid
    out_base = wid * (RPW // 2)

    pltpu.sync_copy(idx_hbm.at[wid], idx_v)
    pltpu.sync_copy(pe_hbm, pe_v)

    @plsc.parallel_loop(0, NCHUNK, step=1)
    def _prep(jr):
        for k in range(CHUNK // LANES):
            sl = pl.ds(k * LANES, LANES)
            v = idx_v[jr, sl]
            pair_v[jr, sl] = jnp.right_shift(v, 1)
            par_v[jr, sl] = jnp.left_shift(jnp.bitwise_and(v, 1), 6)

    for b in range(NBUF - 1):
        pltpu.async_copy(table_hbm.at[pair_v.at[b]], bufs[b], sg[b])

    def group(g, carry):
        for b in range(NBUF):
            j = g * NBUF + b
            buf, obuf = bufs[b], obufs[b]
            pltpu.make_async_copy(table_hbm.at[pair_v.at[j]], buf, sg[b]).wait()

            def _retire():
                pltpu.make_async_copy(
                    obuf, out_hbm.at[pl.ds(0, CHUNK // 2)], so[b]).wait()

            pl.when(g >= 1)(_retire)

            p0 = (j * CHUNK) % SEQ_LEN

            @plsc.parallel_loop(0, CHUNK, step=LANES)
            def _row(i):
                parv = par_v[j, pl.ds(i, LANES)]
                i2 = i // 2
                for r in range(LANES):
                    off = parv[r]
                    for k in range(EMBED_DIM // LANES):
                        src = buf[i + r, pl.ds(off + k * LANES, LANES)]
                        pe_part = pe_v[p0 + i + r, pl.ds(k * LANES, LANES)]
                        dst = pl.ds((r % 2) * EMBED_DIM + k * LANES, LANES)
                        obuf[i2 + r // 2, dst] = src * SCALE + pe_part

            pltpu.async_copy(
                obuf,
                out_hbm.at[pl.ds(out_base + j * (CHUNK // 2), CHUNK // 2)],
                so[b])

            bp = (b - 1) % NBUF

            def _refill():
                pltpu.async_copy(
                    table_hbm.at[pair_v.at[j + NBUF - 1]], bufs[bp], sg[bp])

            pl.when(j + NBUF - 1 < NCHUNK)(_refill)
        return carry

    lax.fori_loop(0, NCHUNK // NBUF, group, 0)

    for b in range(NBUF):
        pltpu.make_async_copy(
            obufs[b], out_hbm.at[pl.ds(0, CHUNK // 2)], so[b]).wait()


_emb = pl.kernel(
    _sc_body,
    out_type=jax.ShapeDtypeStruct((ROWS // 2, 128), jnp.float32),
    mesh=plsc.VectorSubcoreMesh(
        core_axis_name="c", subcore_axis_name="s",
        num_cores=NC, num_subcores=NS,
    ),
    scratch_types=(
        [pltpu.VMEM((NCHUNK, CHUNK), jnp.int32),
         pltpu.VMEM((NCHUNK, CHUNK), jnp.int32),
         pltpu.VMEM((NCHUNK, CHUNK), jnp.int32),
         pltpu.VMEM((PE_EXT, EMBED_DIM), jnp.float32)]
        + [pltpu.VMEM((CHUNK, 128), jnp.float32) for _ in range(NBUF)]
        + [pltpu.VMEM((CHUNK // 2, 128), jnp.float32) for _ in range(NBUF)]
        + [pltpu.SemaphoreType.DMA for _ in range(2 * NBUF)]
    ),
)


def kernel(input_seq, embed_weight):
    idx = input_seq.astype(jnp.int32).reshape(NW, NCHUNK, CHUNK)
    table2 = embed_weight.reshape(500000, 128)
    out = _emb(idx, _positional_table(), table2)
    return out.reshape(BATCH, SEQ_LEN, EMBED_DIM)


# speedup vs baseline: 1.1050x; 1.1050x over previous
import math

import jax
import jax.numpy as jnp
from jax import lax
from jax.experimental import pallas as pl
from jax.experimental.pallas import tpu as pltpu
from jax.experimental.pallas import tpu_sc as plsc

EMBED_DIM = 64
SEQ_LEN = 200
BATCH = 1024
ROWS = BATCH * SEQ_LEN
NC, NS, LANES = 2, 16, 16
NW = NC * NS
RPW = ROWS // NW
CHUNK = 64
NCHUNK = RPW // CHUNK
NBUF = 2
PE_EXT = 256
PAIR_DIM = 2 * EMBED_DIM
SCALE = math.sqrt(EMBED_DIM)


def _positional_table():
    pe_len = SEQ_LEN * 2
    pos = jnp.arange(pe_len, dtype=jnp.float32)[:, None]
    i = jnp.arange(0, EMBED_DIM, 2, dtype=jnp.float32)[None, :]
    sin_part = jnp.sin(pos / jnp.power(10000.0, 2.0 * i / EMBED_DIM))
    cos_part = jnp.cos(pos / jnp.power(10000.0, 2.0 * (i + 1.0) / EMBED_DIM))
    pe = jnp.zeros((pe_len, EMBED_DIM), dtype=jnp.float32)
    pe = pe.at[:, 0::2].set(sin_part)
    pe = pe.at[:, 1::2].set(cos_part)
    pe = pe[:SEQ_LEN]
    return jnp.concatenate([pe, pe[: PE_EXT - SEQ_LEN]], axis=0)


def _sc_body(idx_hbm, pe_hbm, pm_hbm, table_hbm, out_hbm, idx_v, pe_v, *rest):
    bufs = rest[:NBUF]
    obufs = rest[NBUF:2 * NBUF]
    islots = rest[2 * NBUF:3 * NBUF]
    pmslots = rest[3 * NBUF:4 * NBUF]
    sg = rest[4 * NBUF:5 * NBUF]
    sp = rest[5 * NBUF:6 * NBUF]
    so = rest[6 * NBUF:]
    cid = lax.axis_index("c")
    sid = lax.axis_index("s")
    wid = sid * NC + cid
    out_base = wid * RPW

    pltpu.sync_copy(idx_hbm.at[wid], idx_v)
    pltpu.sync_copy(pe_hbm, pe_v)

    def _fetch(j, b):
        for k in range(CHUNK // LANES):
            sl = pl.ds(k * LANES, LANES)
            islots[b][sl] = idx_v[j, sl]
        pltpu.async_copy(table_hbm.at[islots[b]], bufs[b], sg[b])
        pltpu.async_copy(pm_hbm.at[wid].at[j], pmslots[b], sp[b])

    for b in range(NBUF - 1):
        _fetch(b, b)

    def group(g, carry):
        for b in range(NBUF):
            j = g * NBUF + b
            buf, obuf, pm = bufs[b], obufs[b], pmslots[b]
            pltpu.make_async_copy(table_hbm.at[islots[b]], buf, sg[b]).wait()
            pltpu.make_async_copy(pm_hbm.at[wid].at[j], pm, sp[b]).wait()

            bp = (b + NBUF - 1) % NBUF

            def _refill():
                _fetch(j + NBUF - 1, bp)

            pl.when(j + NBUF - 1 < NCHUNK)(_refill)

            def _retire():
                pltpu.make_async_copy(
                    obuf, out_hbm.at[pl.ds(0, CHUNK)], so[b]).wait()

            pl.when(g >= 1)(_retire)

            p0 = (j * CHUNK) % SEQ_LEN

            @plsc.parallel_loop(0, CHUNK, step=1, unroll=4)
            def _row(i):
                m = pm[i, pl.ds(0, LANES)]
                for k in range(EMBED_DIM // LANES):
                    sl = pl.ds(k * LANES, LANES)
                    lo = buf[i, sl]
                    hi = buf[i, pl.ds(EMBED_DIM + k * LANES, LANES)]
                    obuf[i, sl] = ((lo + (hi - lo) * m) * SCALE
                                   + pe_v[p0 + i, sl])

            pltpu.async_copy(
                obuf, out_hbm.at[pl.ds(out_base + j * CHUNK, CHUNK)], so[b])
        return carry

    lax.fori_loop(0, NCHUNK // NBUF, group, 0)

    for b in range(NBUF):
        pltpu.make_async_copy(
            obufs[b], out_hbm.at[pl.ds(0, CHUNK)], so[b]).wait()


_emb = pl.kernel(
    _sc_body,
    out_type=jax.ShapeDtypeStruct((ROWS, EMBED_DIM), jnp.float32),
    mesh=plsc.VectorSubcoreMesh(
        core_axis_name="c", subcore_axis_name="s",
        num_cores=NC, num_subcores=NS,
    ),
    scratch_types=(
        [pltpu.VMEM((NCHUNK, CHUNK), jnp.int32),
         pltpu.VMEM((PE_EXT, EMBED_DIM), jnp.float32)]
        + [pltpu.VMEM((CHUNK, PAIR_DIM), jnp.float32) for _ in range(NBUF)]
        + [pltpu.VMEM((CHUNK, EMBED_DIM), jnp.float32) for _ in range(NBUF)]
        + [pltpu.VMEM((CHUNK,), jnp.int32) for _ in range(NBUF)]
        + [pltpu.VMEM((CHUNK, LANES), jnp.float32) for _ in range(NBUF)]
        + [pltpu.SemaphoreType.DMA for _ in range(3 * NBUF)]
    ),
)


def kernel(input_seq, embed_weight):
    idx = input_seq.astype(jnp.int32).reshape(NW, NCHUNK, CHUNK)
    pair = idx >> 1
    parity = jnp.broadcast_to(
        (idx & 1).astype(jnp.float32)[..., None], (NW, NCHUNK, CHUNK, LANES))
    tview = embed_weight.reshape(embed_weight.shape[0] // 2, PAIR_DIM)
    out = _emb(pair, _positional_table(), parity, tview)
    return out.reshape(BATCH, SEQ_LEN, EMBED_DIM)


# speedup vs baseline: 1.1408x; 1.0324x over previous
import math

import jax
import jax.numpy as jnp
from jax import lax
from jax.experimental import pallas as pl
from jax.experimental.pallas import tpu as pltpu
from jax.experimental.pallas import tpu_sc as plsc

EMBED_DIM = 64
SEQ_LEN = 200
BATCH = 1024
ROWS = BATCH * SEQ_LEN
NC, NS, LANES = 2, 16, 16
NW = NC * NS
RPW = ROWS // NW
CHUNK = 32
NCHUNK = RPW // CHUNK
NBUF = 4
PE_EXT = 224
PAIR_DIM = 2 * EMBED_DIM
SCALE = math.sqrt(EMBED_DIM)


def _positional_table():
    pe_len = SEQ_LEN * 2
    pos = jnp.arange(pe_len, dtype=jnp.float32)[:, None]
    i = jnp.arange(0, EMBED_DIM, 2, dtype=jnp.float32)[None, :]
    sin_part = jnp.sin(pos / jnp.power(10000.0, 2.0 * i / EMBED_DIM))
    cos_part = jnp.cos(pos / jnp.power(10000.0, 2.0 * (i + 1.0) / EMBED_DIM))
    pe = jnp.zeros((pe_len, EMBED_DIM), dtype=jnp.float32)
    pe = pe.at[:, 0::2].set(sin_part)
    pe = pe.at[:, 1::2].set(cos_part)
    pe = pe[:SEQ_LEN]
    return jnp.concatenate([pe, pe[: PE_EXT - SEQ_LEN]], axis=0)


def _sc_body(idx_hbm, pe_hbm, pm_hbm, table_hbm, out_hbm, idx_v, pe_v, *rest):
    bufs = rest[:NBUF]
    obufs = rest[NBUF:2 * NBUF]
    islots = rest[2 * NBUF:3 * NBUF]
    pmslots = rest[3 * NBUF:4 * NBUF]
    sg = rest[4 * NBUF:5 * NBUF]
    sp = rest[5 * NBUF:6 * NBUF]
    so = rest[6 * NBUF:]
    cid = lax.axis_index("c")
    sid = lax.axis_index("s")
    wid = sid * NC + cid
    out_base = wid * RPW

    pltpu.sync_copy(idx_hbm.at[wid], idx_v)
    pltpu.sync_copy(pe_hbm, pe_v)

    def _fetch(j, b):
        for k in range(CHUNK // LANES):
            sl = pl.ds(k * LANES, LANES)
            islots[b][sl] = idx_v[j, sl]
        pltpu.async_copy(table_hbm.at[islots[b]], bufs[b], sg[b])
        pltpu.async_copy(pm_hbm.at[wid].at[j], pmslots[b], sp[b])

    for b in range(NBUF - 1):
        _fetch(b, b)

    def group(g, carry):
        for b in range(NBUF):
            j = g * NBUF + b
            buf, obuf, pm = bufs[b], obufs[b], pmslots[b]
            pltpu.make_async_copy(table_hbm.at[islots[b]], buf, sg[b]).wait()
            pltpu.make_async_copy(pm_hbm.at[wid].at[j], pm, sp[b]).wait()

            bp = (b + NBUF - 1) % NBUF

            def _refill():
                _fetch(j + NBUF - 1, bp)

            pl.when(j + NBUF - 1 < NCHUNK)(_refill)

            def _retire():
                pltpu.make_async_copy(
                    obuf, out_hbm.at[pl.ds(0, CHUNK)], so[b]).wait()

            pl.when(g >= 1)(_retire)

            p0 = (j * CHUNK) % SEQ_LEN

            @plsc.parallel_loop(0, CHUNK, step=1, unroll=4)
            def _row(i):
                m = pm[i, pl.ds(0, LANES)]
                for k in range(EMBED_DIM // LANES):
                    sl = pl.ds(k * LANES, LANES)
                    lo = buf[i, sl]
                    hi = buf[i, pl.ds(EMBED_DIM + k * LANES, LANES)]
                    obuf[i, sl] = ((lo + (hi - lo) * m) * SCALE
                                   + pe_v[p0 + i, sl])

            pltpu.async_copy(
                obuf, out_hbm.at[pl.ds(out_base + j * CHUNK, CHUNK)], so[b])
        return carry

    lax.fori_loop(0, NCHUNK // NBUF, group, 0)

    for b in range(NBUF):
        pltpu.make_async_copy(
            obufs[b], out_hbm.at[pl.ds(0, CHUNK)], so[b]).wait()


_emb = pl.kernel(
    _sc_body,
    out_type=jax.ShapeDtypeStruct((ROWS, EMBED_DIM), jnp.float32),
    mesh=plsc.VectorSubcoreMesh(
        core_axis_name="c", subcore_axis_name="s",
        num_cores=NC, num_subcores=NS,
    ),
    scratch_types=(
        [pltpu.VMEM((NCHUNK, CHUNK), jnp.int32),
         pltpu.VMEM((PE_EXT, EMBED_DIM), jnp.float32)]
        + [pltpu.VMEM((CHUNK, PAIR_DIM), jnp.float32) for _ in range(NBUF)]
        + [pltpu.VMEM((CHUNK, EMBED_DIM), jnp.float32) for _ in range(NBUF)]
        + [pltpu.VMEM((CHUNK,), jnp.int32) for _ in range(NBUF)]
        + [pltpu.VMEM((CHUNK, LANES), jnp.float32) for _ in range(NBUF)]
        + [pltpu.SemaphoreType.DMA for _ in range(3 * NBUF)]
    ),
)


def kernel(input_seq, embed_weight):
    idx = input_seq.astype(jnp.int32).reshape(NW, NCHUNK, CHUNK)
    pair = idx >> 1
    parity = jnp.broadcast_to(
        (idx & 1).astype(jnp.float32)[..., None], (NW, NCHUNK, CHUNK, LANES))
    tview = embed_weight.reshape(embed_weight.shape[0] // 2, PAIR_DIM)
    out = _emb(pair, _positional_table(), parity, tview)
    return out.reshape(BATCH, SEQ_LEN, EMBED_DIM)


# speedup vs baseline: 1.1424x; 1.0014x over previous
import math

import jax
import jax.numpy as jnp
from jax import lax
from jax.experimental import pallas as pl
from jax.experimental.pallas import tpu as pltpu
from jax.experimental.pallas import tpu_sc as plsc

EMBED_DIM = 64
SEQ_LEN = 200
BATCH = 1024
ROWS = BATCH * SEQ_LEN
NC, NS, LANES = 2, 16, 16
NW = NC * NS
RPW = ROWS // NW
CHUNK = 32
NCHUNK = RPW // CHUNK
NBUF = 5
PE_EXT = 224
PAIR_DIM = 2 * EMBED_DIM
SCALE = math.sqrt(EMBED_DIM)


def _positional_table():
    pe_len = SEQ_LEN * 2
    pos = jnp.arange(pe_len, dtype=jnp.float32)[:, None]
    i = jnp.arange(0, EMBED_DIM, 2, dtype=jnp.float32)[None, :]
    sin_part = jnp.sin(pos / jnp.power(10000.0, 2.0 * i / EMBED_DIM))
    cos_part = jnp.cos(pos / jnp.power(10000.0, 2.0 * (i + 1.0) / EMBED_DIM))
    pe = jnp.zeros((pe_len, EMBED_DIM), dtype=jnp.float32)
    pe = pe.at[:, 0::2].set(sin_part)
    pe = pe.at[:, 1::2].set(cos_part)
    pe = pe[:SEQ_LEN]
    return jnp.concatenate([pe, pe[: PE_EXT - SEQ_LEN]], axis=0)


def _sc_body(idx_hbm, pe_hbm, pm_hbm, table_hbm, out_hbm, idx_v, pe_v, *rest):
    bufs = rest[:NBUF]
    obufs = rest[NBUF:2 * NBUF]
    islots = rest[2 * NBUF:3 * NBUF]
    pmslots = rest[3 * NBUF:4 * NBUF]
    sg = rest[4 * NBUF:5 * NBUF]
    sp = rest[5 * NBUF:6 * NBUF]
    so = rest[6 * NBUF:]
    cid = lax.axis_index("c")
    sid = lax.axis_index("s")
    wid = sid * NC + cid
    out_base = wid * RPW

    pltpu.sync_copy(idx_hbm.at[wid], idx_v)
    pltpu.sync_copy(pe_hbm, pe_v)

    def _fetch(j, b):
        for k in range(CHUNK // LANES):
            sl = pl.ds(k * LANES, LANES)
            islots[b][sl] = idx_v[j, sl]
        pltpu.async_copy(table_hbm.at[islots[b]], bufs[b], sg[b])
        pltpu.async_copy(pm_hbm.at[wid].at[j], pmslots[b], sp[b])

    for b in range(NBUF - 1):
        _fetch(b, b)

    def group(g, carry):
        for b in range(NBUF):
            j = g * NBUF + b
            buf, obuf, pm = bufs[b], obufs[b], pmslots[b]
            pltpu.make_async_copy(table_hbm.at[islots[b]], buf, sg[b]).wait()
            pltpu.make_async_copy(pm_hbm.at[wid].at[j], pm, sp[b]).wait()

            bp = (b + NBUF - 1) % NBUF

            def _refill():
                _fetch(j + NBUF - 1, bp)

            pl.when(j + NBUF - 1 < NCHUNK)(_refill)

            def _retire():
                pltpu.make_async_copy(
                    obuf, out_hbm.at[pl.ds(0, CHUNK)], so[b]).wait()

            pl.when(g >= 1)(_retire)

            p0 = (j * CHUNK) % SEQ_LEN

            @plsc.parallel_loop(0, CHUNK, step=1, unroll=4)
            def _row(i):
                m = pm[i, pl.ds(0, LANES)]
                for k in range(EMBED_DIM // LANES):
                    sl = pl.ds(k * LANES, LANES)
                    lo = buf[i, sl]
                    hi = buf[i, pl.ds(EMBED_DIM + k * LANES, LANES)]
                    obuf[i, sl] = ((lo + (hi - lo) * m) * SCALE
                                   + pe_v[p0 + i, sl])

            pltpu.async_copy(
                obuf, out_hbm.at[pl.ds(out_base + j * CHUNK, CHUNK)], so[b])
        return carry

    lax.fori_loop(0, NCHUNK // NBUF, group, 0)

    for b in range(NBUF):
        pltpu.make_async_copy(
            obufs[b], out_hbm.at[pl.ds(0, CHUNK)], so[b]).wait()


_emb = pl.kernel(
    _sc_body,
    out_type=jax.ShapeDtypeStruct((ROWS, EMBED_DIM), jnp.float32),
    mesh=plsc.VectorSubcoreMesh(
        core_axis_name="c", subcore_axis_name="s",
        num_cores=NC, num_subcores=NS,
    ),
    scratch_types=(
        [pltpu.VMEM((NCHUNK, CHUNK), jnp.int32),
         pltpu.VMEM((PE_EXT, EMBED_DIM), jnp.float32)]
        + [pltpu.VMEM((CHUNK, PAIR_DIM), jnp.float32) for _ in range(NBUF)]
        + [pltpu.VMEM((CHUNK, EMBED_DIM), jnp.float32) for _ in range(NBUF)]
        + [pltpu.VMEM((CHUNK,), jnp.int32) for _ in range(NBUF)]
        + [pltpu.VMEM((CHUNK, LANES), jnp.float32) for _ in range(NBUF)]
        + [pltpu.SemaphoreType.DMA for _ in range(3 * NBUF)]
    ),
)


def kernel(input_seq, embed_weight):
    idx = input_seq.astype(jnp.int32).reshape(NW, NCHUNK, CHUNK)
    pair = idx >> 1
    parity = jnp.broadcast_to(
        (idx & 1).astype(jnp.float32)[..., None], (NW, NCHUNK, CHUNK, LANES))
    tview = embed_weight.reshape(embed_weight.shape[0] // 2, PAIR_DIM)
    out = _emb(pair, _positional_table(), parity, tview)
    return out.reshape(BATCH, SEQ_LEN, EMBED_DIM)
